# Initial kernel scaffold; baseline (speedup 1.0000x reference)
import jax
import jax.numpy as jnp
from jax import lax
from jax.experimental import pallas as pl
from jax.experimental.pallas import tpu as pltpu
from jax.experimental.pallas import tpu_sc as plsc

N = 10000
E = 320000
D_IN = 128
D_H = 256

NSC = 2
NT = 16
K = 200
NCH = E // NT // K
ROWS_PER_TILE = N // NT

_mesh = plsc.VectorSubcoreMesh(core_axis_name="c", subcore_axis_name="s")


def _deg_body(idx_hbm, ones_hbm, zeros_hbm, out_hbm, idx_v, ones_v, acc):
    c = lax.axis_index("c")
    s = lax.axis_index("s")
    pltpu.sync_copy(idx_hbm.at[c, s], idx_v)
    pltpu.sync_copy(ones_hbm, ones_v)
    pltpu.sync_copy(zeros_hbm, acc.at[pl.ds(s * ROWS_PER_TILE, ROWS_PER_TILE)])
    plsc.subcore_barrier()

    def body(j, carry):
        pltpu.sync_copy(ones_v, acc.at[idx_v.at[j]], add=True)
        return carry

    lax.fori_loop(0, NCH, body, 0)
    plsc.subcore_barrier()
    sl = pl.ds(s * ROWS_PER_TILE, ROWS_PER_TILE)
    pltpu.sync_copy(acc.at[sl], out_hbm.at[c, sl])


_deg_call = pl.kernel(
    _deg_body,
    out_type=jax.ShapeDtypeStruct((NSC, N, 16), jnp.float32),
    mesh=_mesh,
    scratch_types=[
        pltpu.VMEM((NCH, K), jnp.int32),
        pltpu.VMEM((K, 16), jnp.float32),
        pltpu.VMEM_SHARED((N, 16), jnp.float32),
    ],
)


def _mp_body(table_hbm, srcr_hbm, dstr_hbm, zeros_hbm, out_hbm,
             src_v, dst_v, buf, sem, acc):
    c = lax.axis_index("c")
    s = lax.axis_index("s")
    pltpu.sync_copy(srcr_hbm.at[c, s], src_v)
    pltpu.sync_copy(dstr_hbm.at[s], dst_v)
    pltpu.sync_copy(zeros_hbm, acc.at[pl.ds(s * ROWS_PER_TILE, ROWS_PER_TILE)])
    plsc.subcore_barrier()

    def body(j, carry):
        pltpu.async_copy(table_hbm.at[src_v.at[j]], buf, sem).wait()
        pltpu.sync_copy(buf, acc.at[dst_v.at[j]], add=True)
        return carry

    lax.fori_loop(0, NCH, body, 0)
    plsc.subcore_barrier()
    sl = pl.ds(s * ROWS_PER_TILE, ROWS_PER_TILE)
    pltpu.sync_copy(acc.at[sl], out_hbm.at[c, sl])


def _make_mp_call(dc):
    return pl.kernel(
        _mp_body,
        out_type=jax.ShapeDtypeStruct((NSC, N, dc), jnp.float32),
        mesh=_mesh,
        scratch_types=[
            pltpu.VMEM((NCH, K), jnp.int32),
            pltpu.VMEM((NCH, K), jnp.int32),
            pltpu.VMEM((K, dc), jnp.float32),
            pltpu.SemaphoreType.DMA,
            pltpu.VMEM_SHARED((N, dc), jnp.float32),
        ],
    )


_mp_call_64 = _make_mp_call(64)
_mp_call_128 = _make_mp_call(128)


_BLK = 400
_NBLK = N // _BLK


def _rsqrt_clip(deg):
    return lax.rsqrt(jnp.maximum(deg, 1.0))


def _tc_scale_body(x_ref, od_ref, o_ref):
    p = _rsqrt_clip(od_ref[:, 0:1])
    xs = x_ref[...] * p
    o_ref[0] = xs[:, :64]
    o_ref[1] = xs[:, 64:]


def _tc_scale(x, od):
    return pl.pallas_call(
        _tc_scale_body,
        grid=(_NBLK,),
        in_specs=[
            pl.BlockSpec((_BLK, D_IN), lambda i: (i, 0)),
            pl.BlockSpec((_BLK, 16), lambda i: (i, 0)),
        ],
        out_specs=pl.BlockSpec((NSC, _BLK, 64), lambda i: (0, i, 0)),
        out_shape=jax.ShapeDtypeStruct((NSC, N, 64), jnp.float32),
    )(x, od)


def _tc_layer1_body(acc_ref, od_ref, id_ref, w0_ref, b0_ref, o_ref):
    m = jnp.dot(acc_ref[0], w0_ref[:64, :], preferred_element_type=jnp.float32)
    m += jnp.dot(acc_ref[1], w0_ref[64:, :], preferred_element_type=jnp.float32)
    q = _rsqrt_clip(id_ref[:, 0:1])
    h = jnp.maximum(q * m + b0_ref[...], 0.0)
    z = h * _rsqrt_clip(od_ref[:, 0:1])
    o_ref[0] = z[:, :128]
    o_ref[1] = z[:, 128:]


def _tc_layer1(acc1, od, idg, w0, b0):
    return pl.pallas_call(
        _tc_layer1_body,
        grid=(_NBLK,),
        in_specs=[
            pl.BlockSpec((NSC, _BLK, 64), lambda i: (0, i, 0)),
            pl.BlockSpec((_BLK, 16), lambda i: (i, 0)),
            pl.BlockSpec((_BLK, 16), lambda i: (i, 0)),
            pl.BlockSpec((D_IN, D_H), lambda i: (0, 0)),
            pl.BlockSpec((1, D_H), lambda i: (0, 0)),
        ],
        out_specs=pl.BlockSpec((NSC, _BLK, 128), lambda i: (0, i, 0)),
        out_shape=jax.ShapeDtypeStruct((NSC, N, 128), jnp.float32),
    )(acc1, od, idg, w0, b0)


def _tc_head_body(acc_ref, id_ref, w1_ref, b1_ref, ap_ref, wl_ref, bl_ref,
                  o_ref, accum):
    j = pl.program_id(0)

    @pl.when(j == 0)
    def _():
        accum[...] = jnp.zeros_like(accum)

    m = jnp.dot(acc_ref[0], w1_ref[:128, :], preferred_element_type=jnp.float32)
    m += jnp.dot(acc_ref[1], w1_ref[128:, :], preferred_element_type=jnp.float32)
    q = _rsqrt_clip(id_ref[:, 0:1])
    h = jnp.maximum(q * m + b1_ref[...], 0.0)
    accum[...] += jnp.sum(h, axis=0, keepdims=True)

    @pl.when(j == _NBLK - 1)
    def _():
        g = accum[...] * (1.0 / N)
        g = jnp.where(g >= 0.0, g, ap_ref[0, 0] * g)
        v = jnp.dot(g, wl_ref[...], preferred_element_type=jnp.float32)
        o_ref[...] = 1.0 / (1.0 + jnp.exp(-(v + bl_ref[...])))


def _tc_head(acc2, idg, w1, b1, ap, wl, bl):
    return pl.pallas_call(
        _tc_head_body,
        grid=(_NBLK,),
        in_specs=[
            pl.BlockSpec((NSC, _BLK, 128), lambda i: (0, i, 0)),
            pl.BlockSpec((_BLK, 16), lambda i: (i, 0)),
            pl.BlockSpec((D_H, D_H), lambda i: (0, 0)),
            pl.BlockSpec((1, D_H), lambda i: (0, 0)),
            pl.BlockSpec((1, 1), lambda i: (0, 0)),
            pl.BlockSpec((D_H, 1), lambda i: (0, 0)),
            pl.BlockSpec((1, 1), lambda i: (0, 0)),
        ],
        out_specs=pl.BlockSpec((1, 1), lambda i: (0, 0)),
        out_shape=jax.ShapeDtypeStruct((1, 1), jnp.float32),
        scratch_shapes=[pltpu.VMEM((1, D_H), jnp.float32)],
    )(acc2, idg, w1, b1, ap, wl, bl)


@jax.jit
def kernel(x, edge_index, W0, b0, W1, b1, a_prelu, Wl, bl):
    src = edge_index[0]
    dst = edge_index[1]

    srcr = jnp.stack([src, src + N]).reshape(NSC, NT, NCH, K)
    dstr = dst.reshape(NT, NCH, K)
    degidx = jnp.stack([src, dst]).reshape(NSC, NT, NCH, K)

    ones16 = jnp.ones((K, 16), jnp.float32)
    zeros16 = jnp.zeros((ROWS_PER_TILE, 16), jnp.float32)
    zeros64 = jnp.zeros((ROWS_PER_TILE, 64), jnp.float32)
    zeros128 = jnp.zeros((ROWS_PER_TILE, 128), jnp.float32)

    deg = _deg_call(degidx, ones16, zeros16)
    od = deg[0]
    idg = deg[1]

    xs = _tc_scale(x, od)
    acc1 = _mp_call_64(xs.reshape(NSC * N, 64), srcr, dstr, zeros64)
    z = _tc_layer1(acc1, od, idg, W0, b0.reshape(1, D_H))
    acc2 = _mp_call_128(z.reshape(NSC * N, 128), srcr, dstr, zeros128)
    out = _tc_head(acc2, idg, W1, b1.reshape(1, D_H),
                   a_prelu.reshape(1, 1), Wl, bl.reshape(1, 1))
    return out


# speedup vs baseline: 15.2224x; 15.2224x over previous
import functools

import jax
import jax.numpy as jnp
from jax import lax
from jax.experimental import pallas as pl
from jax.experimental.pallas import tpu as pltpu
from jax.experimental.pallas import tpu_sc as plsc

N = 10000
E = 320000
D_IN = 128
D_H = 256

NSC = 2
NT = 16
K = 200
NPAD = 10240
RPT = NPAD // NT
EPT = E // (NSC * NT)


@functools.lru_cache(maxsize=None)
def _get_mesh():
    return plsc.VectorSubcoreMesh(core_axis_name="c", subcore_axis_name="s")


def _deg_body(src_hbm, dst_hbm, out_hbm, sidx, didx, od_l, id_l, rbuf, obuf, sh):
    c = lax.axis_index("c")
    s = lax.axis_index("s")
    pltpu.sync_copy(src_hbm.at[c, s], sidx)
    pltpu.sync_copy(dst_hbm.at[c, s], didx)

    zeros = jnp.zeros((16,), jnp.float32)
    ones = jnp.ones((16,), jnp.float32)

    def zbody(i, carry):
        od_l[pl.ds(i * 16, 16)] = zeros
        id_l[pl.ds(i * 16, 16)] = zeros
        return carry

    lax.fori_loop(0, NPAD // 16, zbody, 0)

    def hbody(i, carry):
        plsc.addupdate_scatter(od_l, [sidx[pl.ds(i * 16, 16)]], ones)
        plsc.addupdate_scatter(id_l, [didx[pl.ds(i * 16, 16)]], ones)
        return carry

    lax.fori_loop(0, EPT // 16, hbody, 0)

    pltpu.sync_copy(od_l, sh.at[s, 0])
    pltpu.sync_copy(id_l, sh.at[s, 1])
    plsc.subcore_barrier()

    for q in range(2):
        for w in range(NT):
            pltpu.sync_copy(sh.at[w, q, pl.ds(s * RPT, RPT)], rbuf.at[w])

        def rbody(i, carry):
            acc = rbuf[0, pl.ds(i * 16, 16)]
            for w in range(1, NT):
                acc += rbuf[w, pl.ds(i * 16, 16)]
            obuf[pl.ds(i * 16, 16)] = acc
            return carry

        lax.fori_loop(0, RPT // 16, rbody, 0)
        pltpu.sync_copy(obuf, out_hbm.at[c, q, s])


@functools.lru_cache(maxsize=None)
def _get_deg_call():
    return pl.kernel(
        _deg_body,
        out_type=jax.ShapeDtypeStruct((NSC, 2, NT, RPT), jnp.float32),
        mesh=_get_mesh(),
        compiler_params=pltpu.CompilerParams(needs_layout_passes=False),
        scratch_types=[
            pltpu.VMEM((EPT,), jnp.int32),
            pltpu.VMEM((EPT,), jnp.int32),
            pltpu.VMEM((NPAD,), jnp.float32),
            pltpu.VMEM((NPAD,), jnp.float32),
            pltpu.VMEM((NT, RPT), jnp.float32),
            pltpu.VMEM((RPT,), jnp.float32),
            pltpu.VMEM_SHARED((NT, 2, NPAD), jnp.float32),
        ],
    )


NCH = E // (NSC * NT * K)


def _mp_body(table_hbm, srcr_hbm, dstr_hbm, zeros_hbm, out_hbm,
             src_v, dst_v, buf, sem, acc):
    c = lax.axis_index("c")
    s = lax.axis_index("s")
    pltpu.sync_copy(srcr_hbm.at[c, s], src_v)
    pltpu.sync_copy(dstr_hbm.at[c, s], dst_v)
    pltpu.sync_copy(zeros_hbm, acc.at[pl.ds(s * RPT, RPT)])
    plsc.subcore_barrier()

    def body(j, carry):
        pltpu.async_copy(table_hbm.at[src_v.at[j]], buf, sem).wait()
        pltpu.sync_copy(buf, acc.at[dst_v.at[j]], add=True)
        return carry

    lax.fori_loop(0, NCH, body, 0)
    plsc.subcore_barrier()
    sl = pl.ds(s * RPT, RPT)
    pltpu.sync_copy(acc.at[sl], out_hbm.at[c, sl])


@functools.lru_cache(maxsize=None)
def _get_mp_call():
    return pl.kernel(
        _mp_body,
        out_type=jax.ShapeDtypeStruct((NSC, NPAD, 128), jnp.float32),
        mesh=_get_mesh(),
        compiler_params=pltpu.CompilerParams(use_tc_tiling_on_sc=False),
        scratch_types=[
            pltpu.VMEM((NCH, K), jnp.int32),
            pltpu.VMEM((NCH, K), jnp.int32),
            pltpu.VMEM((K, 128), jnp.float32),
            pltpu.SemaphoreType.DMA,
            pltpu.VMEM_SHARED((NPAD, 128), jnp.float32),
        ],
    )


_BLK = 400
_NBLK = N // _BLK


def _rsqrt_clip(deg):
    return lax.rsqrt(jnp.maximum(deg, 1.0))


def _tc_scale_body(x_ref, od_ref, o_ref):
    p = _rsqrt_clip(od_ref[0] + od_ref[1])
    o_ref[...] = x_ref[...] * p


def _tc_scale(x, od2):
    return pl.pallas_call(
        _tc_scale_body,
        grid=(_NBLK,),
        in_specs=[
            pl.BlockSpec((_BLK, D_IN), lambda i: (i, 0)),
            pl.BlockSpec((NSC, _BLK, 1), lambda i: (0, i, 0)),
        ],
        out_specs=pl.BlockSpec((_BLK, D_IN), lambda i: (i, 0)),
        out_shape=jax.ShapeDtypeStruct((N, D_IN), jnp.float32),
    )(x, od2)


def _tc_layer1_body(acc_ref, od_ref, id_ref, w0_ref, b0_ref, o0_ref, o1_ref):
    a = acc_ref[0] + acc_ref[1]
    m = jnp.dot(a, w0_ref[...], preferred_element_type=jnp.float32)
    q = _rsqrt_clip(id_ref[0] + id_ref[1])
    h = jnp.maximum(q * m + b0_ref[...], 0.0)
    z = h * _rsqrt_clip(od_ref[0] + od_ref[1])
    o0_ref[...] = z[:, :128]
    o1_ref[...] = z[:, 128:]


def _tc_layer1(acc1, od2, id2, w0, b0):
    return pl.pallas_call(
        _tc_layer1_body,
        grid=(_NBLK,),
        in_specs=[
            pl.BlockSpec((NSC, _BLK, D_IN), lambda i: (0, i, 0)),
            pl.BlockSpec((NSC, _BLK, 1), lambda i: (0, i, 0)),
            pl.BlockSpec((NSC, _BLK, 1), lambda i: (0, i, 0)),
            pl.BlockSpec((D_IN, D_H), lambda i: (0, 0)),
            pl.BlockSpec((1, D_H), lambda i: (0, 0)),
        ],
        out_specs=[
            pl.BlockSpec((_BLK, 128), lambda i: (i, 0)),
            pl.BlockSpec((_BLK, 128), lambda i: (i, 0)),
        ],
        out_shape=[
            jax.ShapeDtypeStruct((N, 128), jnp.float32),
            jax.ShapeDtypeStruct((N, 128), jnp.float32),
        ],
    )(acc1, od2, id2, w0, b0)


def _tc_head_body(acca_ref, accb_ref, id_ref, w1_ref, b1_ref, ap_ref, wl_ref,
                  bl_ref, o_ref, accum):
    j = pl.program_id(0)

    @pl.when(j == 0)
    def _():
        accum[...] = jnp.zeros_like(accum)

    a = acca_ref[0] + acca_ref[1]
    b = accb_ref[0] + accb_ref[1]
    m = jnp.dot(a, w1_ref[:128, :], preferred_element_type=jnp.float32)
    m += jnp.dot(b, w1_ref[128:, :], preferred_element_type=jnp.float32)
    q = _rsqrt_clip(id_ref[0] + id_ref[1])
    h = jnp.maximum(q * m + b1_ref[...], 0.0)
    accum[...] += jnp.sum(h, axis=0, keepdims=True)

    @pl.when(j == _NBLK - 1)
    def _():
        g = accum[...] * (1.0 / N)
        g = jnp.where(g >= 0.0, g, ap_ref[0, 0] * g)
        v = jnp.dot(g, wl_ref[...], preferred_element_type=jnp.float32)
        o_ref[...] = 1.0 / (1.0 + jnp.exp(-(v + bl_ref[...])))


def _tc_head(acca, accb, id2, w1, b1, ap, wl, bl):
    return pl.pallas_call(
        _tc_head_body,
        grid=(_NBLK,),
        in_specs=[
            pl.BlockSpec((NSC, _BLK, 128), lambda i: (0, i, 0)),
            pl.BlockSpec((NSC, _BLK, 128), lambda i: (0, i, 0)),
            pl.BlockSpec((NSC, _BLK, 1), lambda i: (0, i, 0)),
            pl.BlockSpec((D_H, D_H), lambda i: (0, 0)),
            pl.BlockSpec((1, D_H), lambda i: (0, 0)),
            pl.BlockSpec((1, 1), lambda i: (0, 0)),
            pl.BlockSpec((D_H, 1), lambda i: (0, 0)),
            pl.BlockSpec((1, 1), lambda i: (0, 0)),
        ],
        out_specs=pl.BlockSpec((1, 1), lambda i: (0, 0)),
        out_shape=jax.ShapeDtypeStruct((1, 1), jnp.float32),
        scratch_shapes=[pltpu.VMEM((1, D_H), jnp.float32)],
    )(acca, accb, id2, w1, b1, ap, wl, bl)


@jax.jit
def kernel(x, edge_index, W0, b0, W1, b1, a_prelu, Wl, bl):
    src = edge_index[0]
    dst = edge_index[1]

    srcf = src.reshape(NSC, NT, EPT)
    dstf = dst.reshape(NSC, NT, EPT)
    deg = _get_deg_call()(srcf, dstf)
    od2 = deg[:, 0].reshape(NSC, NPAD, 1)
    id2 = deg[:, 1].reshape(NSC, NPAD, 1)

    srcr = src.reshape(NSC, NT, NCH, K)
    dstr = dst.reshape(NSC, NT, NCH, K)
    zeros128 = jnp.zeros((RPT, 128), jnp.float32)
    mp = _get_mp_call()

    xs = _tc_scale(x, od2)
    acc1 = mp(xs, srcr, dstr, zeros128)
    z0, z1 = _tc_layer1(acc1, od2, id2, W0, b0.reshape(1, D_H))
    acca = mp(z0, srcr, dstr, zeros128)
    accb = mp(z1, srcr, dstr, zeros128)
    out = _tc_head(acca, accb, id2, W1, b1.reshape(1, D_H),
                   a_prelu.reshape(1, 1), Wl, bl.reshape(1, 1))
    return out


# speedup vs baseline: 19.5439x; 1.2839x over previous
import functools

import jax
import jax.numpy as jnp
from jax import lax
from jax.experimental import pallas as pl
from jax.experimental.pallas import tpu as pltpu
from jax.experimental.pallas import tpu_sc as plsc

N = 10000
E = 320000
D_IN = 128
D_H = 256

NSC = 2
NT = 16
K = 100
NPAD = 10240
RPT = NPAD // NT
EPT = E // (NSC * NT)


@functools.lru_cache(maxsize=None)
def _get_mesh():
    return plsc.VectorSubcoreMesh(core_axis_name="c", subcore_axis_name="s")


def _deg_body(src_hbm, dst_hbm, out_hbm, sidx, didx, od_l, id_l, rbuf, obuf, sh):
    c = lax.axis_index("c")
    s = lax.axis_index("s")
    pltpu.sync_copy(src_hbm.at[c, s], sidx)
    pltpu.sync_copy(dst_hbm.at[c, s], didx)

    zeros = jnp.zeros((16,), jnp.float32)
    ones = jnp.ones((16,), jnp.float32)

    def zbody(i, carry):
        od_l[pl.ds(i * 16, 16)] = zeros
        id_l[pl.ds(i * 16, 16)] = zeros
        return carry

    lax.fori_loop(0, NPAD // 16, zbody, 0)

    def hbody(i, carry):
        plsc.addupdate_scatter(od_l, [sidx[pl.ds(i * 16, 16)]], ones)
        plsc.addupdate_scatter(id_l, [didx[pl.ds(i * 16, 16)]], ones)
        return carry

    lax.fori_loop(0, EPT // 16, hbody, 0)

    pltpu.sync_copy(od_l, sh.at[s, 0])
    pltpu.sync_copy(id_l, sh.at[s, 1])
    plsc.subcore_barrier()

    for q in range(2):
        for w in range(NT):
            pltpu.sync_copy(sh.at[w, q, pl.ds(s * RPT, RPT)], rbuf.at[w])

        def rbody(i, carry):
            acc = rbuf[0, pl.ds(i * 16, 16)]
            for w in range(1, NT):
                acc += rbuf[w, pl.ds(i * 16, 16)]
            obuf[pl.ds(i * 16, 16)] = acc
            return carry

        lax.fori_loop(0, RPT // 16, rbody, 0)
        pltpu.sync_copy(obuf, out_hbm.at[c, q, s])


@functools.lru_cache(maxsize=None)
def _get_deg_call():
    return pl.kernel(
        _deg_body,
        out_type=jax.ShapeDtypeStruct((NSC, 2, NT, RPT), jnp.float32),
        mesh=_get_mesh(),
        compiler_params=pltpu.CompilerParams(needs_layout_passes=False),
        scratch_types=[
            pltpu.VMEM((EPT,), jnp.int32),
            pltpu.VMEM((EPT,), jnp.int32),
            pltpu.VMEM((NPAD,), jnp.float32),
            pltpu.VMEM((NPAD,), jnp.float32),
            pltpu.VMEM((NT, RPT), jnp.float32),
            pltpu.VMEM((RPT,), jnp.float32),
            pltpu.VMEM_SHARED((NT, 2, NPAD), jnp.float32),
        ],
    )


NCH = E // (NSC * NT * K)


def _mp_body(table_hbm, srcr_hbm, dstr_hbm, zeros_hbm, out_hbm,
             src_v, dst_v, buf0, buf1, sem0, sem1, acc):
    c = lax.axis_index("c")
    s = lax.axis_index("s")
    pltpu.sync_copy(srcr_hbm.at[c, s], src_v)
    pltpu.sync_copy(dstr_hbm.at[c, s], dst_v)
    pltpu.sync_copy(zeros_hbm, acc.at[pl.ds(s * RPT, RPT)])
    plsc.subcore_barrier()

    pltpu.async_copy(table_hbm.at[src_v.at[0]], buf0, sem0)

    def body(j, carry):
        a = 2 * j
        b = 2 * j + 1
        pltpu.async_copy(table_hbm.at[src_v.at[b]], buf1, sem1)
        pltpu.make_async_copy(table_hbm.at[src_v.at[a]], buf0, sem0).wait()
        pltpu.sync_copy(buf0, acc.at[dst_v.at[a]], add=True)

        @pl.when(j < NCH // 2 - 1)
        def _():
            pltpu.async_copy(table_hbm.at[src_v.at[a + 2]], buf0, sem0)

        pltpu.make_async_copy(table_hbm.at[src_v.at[b]], buf1, sem1).wait()
        pltpu.sync_copy(buf1, acc.at[dst_v.at[b]], add=True)
        return carry

    lax.fori_loop(0, NCH // 2, body, 0)
    plsc.subcore_barrier()
    sl = pl.ds(s * RPT, RPT)
    pltpu.sync_copy(acc.at[sl], out_hbm.at[c, sl])


@functools.lru_cache(maxsize=None)
def _get_mp_call():
    return pl.kernel(
        _mp_body,
        out_type=jax.ShapeDtypeStruct((NSC, NPAD, 128), jnp.float32),
        mesh=_get_mesh(),
        compiler_params=pltpu.CompilerParams(use_tc_tiling_on_sc=False),
        scratch_types=[
            pltpu.VMEM((NCH, K), jnp.int32),
            pltpu.VMEM((NCH, K), jnp.int32),
            pltpu.VMEM((K, 128), jnp.float32),
            pltpu.VMEM((K, 128), jnp.float32),
            pltpu.SemaphoreType.DMA,
            pltpu.SemaphoreType.DMA,
            pltpu.VMEM_SHARED((NPAD, 128), jnp.float32),
        ],
    )


_BLK = 400
_NBLK = N // _BLK


def _rsqrt_clip(deg):
    return lax.rsqrt(jnp.maximum(deg, 1.0))


def _tc_scale_body(x_ref, od_ref, o_ref):
    p = _rsqrt_clip(od_ref[0] + od_ref[1])
    o_ref[...] = x_ref[...] * p


def _tc_scale(x, od2):
    return pl.pallas_call(
        _tc_scale_body,
        grid=(_NBLK,),
        in_specs=[
            pl.BlockSpec((_BLK, D_IN), lambda i: (i, 0)),
            pl.BlockSpec((NSC, _BLK, 1), lambda i: (0, i, 0)),
        ],
        out_specs=pl.BlockSpec((_BLK, D_IN), lambda i: (i, 0)),
        out_shape=jax.ShapeDtypeStruct((N, D_IN), jnp.float32),
    )(x, od2)


def _tc_layer1_body(acc_ref, od_ref, id_ref, w0_ref, b0_ref, o0_ref, o1_ref):
    a = acc_ref[0] + acc_ref[1]
    m = jnp.dot(a, w0_ref[...], preferred_element_type=jnp.float32)
    q = _rsqrt_clip(id_ref[0] + id_ref[1])
    h = jnp.maximum(q * m + b0_ref[...], 0.0)
    z = h * _rsqrt_clip(od_ref[0] + od_ref[1])
    o0_ref[...] = z[:, :128]
    o1_ref[...] = z[:, 128:]


def _tc_layer1(acc1, od2, id2, w0, b0):
    return pl.pallas_call(
        _tc_layer1_body,
        grid=(_NBLK,),
        in_specs=[
            pl.BlockSpec((NSC, _BLK, D_IN), lambda i: (0, i, 0)),
            pl.BlockSpec((NSC, _BLK, 1), lambda i: (0, i, 0)),
            pl.BlockSpec((NSC, _BLK, 1), lambda i: (0, i, 0)),
            pl.BlockSpec((D_IN, D_H), lambda i: (0, 0)),
            pl.BlockSpec((1, D_H), lambda i: (0, 0)),
        ],
        out_specs=[
            pl.BlockSpec((_BLK, 128), lambda i: (i, 0)),
            pl.BlockSpec((_BLK, 128), lambda i: (i, 0)),
        ],
        out_shape=[
            jax.ShapeDtypeStruct((N, 128), jnp.float32),
            jax.ShapeDtypeStruct((N, 128), jnp.float32),
        ],
    )(acc1, od2, id2, w0, b0)


def _tc_head_body(acca_ref, accb_ref, id_ref, w1_ref, b1_ref, ap_ref, wl_ref,
                  bl_ref, o_ref, accum):
    j = pl.program_id(0)

    @pl.when(j == 0)
    def _():
        accum[...] = jnp.zeros_like(accum)

    a = acca_ref[0] + acca_ref[1]
    b = accb_ref[0] + accb_ref[1]
    m = jnp.dot(a, w1_ref[:128, :], preferred_element_type=jnp.float32)
    m += jnp.dot(b, w1_ref[128:, :], preferred_element_type=jnp.float32)
    q = _rsqrt_clip(id_ref[0] + id_ref[1])
    h = jnp.maximum(q * m + b1_ref[...], 0.0)
    accum[...] += jnp.sum(h, axis=0, keepdims=True)

    @pl.when(j == _NBLK - 1)
    def _():
        g = accum[...] * (1.0 / N)
        g = jnp.where(g >= 0.0, g, ap_ref[0, 0] * g)
        v = jnp.dot(g, wl_ref[...], preferred_element_type=jnp.float32)
        o_ref[...] = 1.0 / (1.0 + jnp.exp(-(v + bl_ref[...])))


def _tc_head(acca, accb, id2, w1, b1, ap, wl, bl):
    return pl.pallas_call(
        _tc_head_body,
        grid=(_NBLK,),
        in_specs=[
            pl.BlockSpec((NSC, _BLK, 128), lambda i: (0, i, 0)),
            pl.BlockSpec((NSC, _BLK, 128), lambda i: (0, i, 0)),
            pl.BlockSpec((NSC, _BLK, 1), lambda i: (0, i, 0)),
            pl.BlockSpec((D_H, D_H), lambda i: (0, 0)),
            pl.BlockSpec((1, D_H), lambda i: (0, 0)),
            pl.BlockSpec((1, 1), lambda i: (0, 0)),
            pl.BlockSpec((D_H, 1), lambda i: (0, 0)),
            pl.BlockSpec((1, 1), lambda i: (0, 0)),
        ],
        out_specs=pl.BlockSpec((1, 1), lambda i: (0, 0)),
        out_shape=jax.ShapeDtypeStruct((1, 1), jnp.float32),
        scratch_shapes=[pltpu.VMEM((1, D_H), jnp.float32)],
    )(acca, accb, id2, w1, b1, ap, wl, bl)


@jax.jit
def kernel(x, edge_index, W0, b0, W1, b1, a_prelu, Wl, bl):
    src = edge_index[0]
    dst = edge_index[1]

    srcf = src.reshape(NSC, NT, EPT)
    dstf = dst.reshape(NSC, NT, EPT)
    deg = _get_deg_call()(srcf, dstf)
    od2 = deg[:, 0].reshape(NSC, NPAD, 1)
    id2 = deg[:, 1].reshape(NSC, NPAD, 1)

    srcr = src.reshape(NSC, NT, NCH, K)
    dstr = dst.reshape(NSC, NT, NCH, K)
    zeros128 = jnp.zeros((RPT, 128), jnp.float32)
    mp = _get_mp_call()

    xs = _tc_scale(x, od2)
    acc1 = mp(xs, srcr, dstr, zeros128)
    z0, z1 = _tc_layer1(acc1, od2, id2, W0, b0.reshape(1, D_H))
    acca = mp(z0, srcr, dstr, zeros128)
    accb = mp(z1, srcr, dstr, zeros128)
    out = _tc_head(acca, accb, id2, W1, b1.reshape(1, D_H),
                   a_prelu.reshape(1, 1), Wl, bl.reshape(1, 1))
    return out


# speedup vs baseline: 21.8734x; 1.1192x over previous
import functools

import jax
import jax.numpy as jnp
from jax import lax
from jax.experimental import pallas as pl
from jax.experimental.pallas import tpu as pltpu
from jax.experimental.pallas import tpu_sc as plsc

N = 10000
E = 320000
D_IN = 128
D_H = 256

NSC = 2
NT = 16
K = 100
NPAD = 10240
RPT = NPAD // NT
EPT = E // (NSC * NT)


@functools.lru_cache(maxsize=None)
def _get_mesh():
    return plsc.VectorSubcoreMesh(core_axis_name="c", subcore_axis_name="s")


def _deg_body(src_hbm, dst_hbm, ood_hbm, oid_hbm, sidx, didx, od_l, id_l, rbuf,
              obuf, sh):
    c = lax.axis_index("c")
    s = lax.axis_index("s")
    pltpu.sync_copy(src_hbm.at[c, s], sidx)
    pltpu.sync_copy(dst_hbm.at[c, s], didx)

    zeros = jnp.zeros((16,), jnp.float32)
    ones = jnp.ones((16,), jnp.float32)

    def zbody(i, carry):
        od_l[pl.ds(i * 16, 16)] = zeros
        id_l[pl.ds(i * 16, 16)] = zeros
        return carry

    lax.fori_loop(0, NPAD // 16, zbody, 0)

    def hbody(i, carry):
        plsc.addupdate_scatter(od_l, [sidx[pl.ds(i * 16, 16)]], ones)
        plsc.addupdate_scatter(id_l, [didx[pl.ds(i * 16, 16)]], ones)
        return carry

    lax.fori_loop(0, EPT // 16, hbody, 0)

    pltpu.sync_copy(od_l, sh.at[s, 0])
    pltpu.sync_copy(id_l, sh.at[s, 1])
    plsc.subcore_barrier()

    for q, out_hbm in ((0, ood_hbm), (1, oid_hbm)):
        for w in range(NT):
            pltpu.sync_copy(sh.at[w, q, pl.ds(s * RPT, RPT)], rbuf.at[w])

        def rbody(i, carry):
            acc = rbuf[0, pl.ds(i * 16, 16)]
            for w in range(1, NT):
                acc += rbuf[w, pl.ds(i * 16, 16)]
            obuf[pl.ds(i * 16, 16)] = acc
            return carry

        lax.fori_loop(0, RPT // 16, rbody, 0)
        pltpu.sync_copy(obuf, out_hbm.at[c, s])


@functools.lru_cache(maxsize=None)
def _get_deg_call():
    return pl.kernel(
        _deg_body,
        out_type=[jax.ShapeDtypeStruct((NSC, NT, RPT), jnp.float32),
                  jax.ShapeDtypeStruct((NSC, NT, RPT), jnp.float32)],
        mesh=_get_mesh(),
        compiler_params=pltpu.CompilerParams(needs_layout_passes=False),
        scratch_types=[
            pltpu.VMEM((EPT,), jnp.int32),
            pltpu.VMEM((EPT,), jnp.int32),
            pltpu.VMEM((NPAD,), jnp.float32),
            pltpu.VMEM((NPAD,), jnp.float32),
            pltpu.VMEM((NT, RPT), jnp.float32),
            pltpu.VMEM((RPT,), jnp.float32),
            pltpu.VMEM_SHARED((NT, 2, NPAD), jnp.float32),
        ],
    )


NCH = 100


def _mp_body(table_hbm, srcr_hbm, dstr_hbm, zeros_hbm, out_hbm,
             src_v, dst_v, buf0, buf1, sem0, sem1, acc, *, rounds):
    c = lax.axis_index("c")
    s = lax.axis_index("s")
    pltpu.sync_copy(zeros_hbm, acc.at[pl.ds(s * RPT, RPT)])
    plsc.subcore_barrier()

    for r in range(rounds):
        pltpu.sync_copy(srcr_hbm.at[c, s, r], src_v)
        pltpu.sync_copy(dstr_hbm.at[c, s, r], dst_v)
        pltpu.async_copy(table_hbm.at[src_v.at[0]], buf0, sem0)

        def body(j, carry):
            a = 2 * j
            b = 2 * j + 1
            pltpu.async_copy(table_hbm.at[src_v.at[b]], buf1, sem1)
            pltpu.make_async_copy(table_hbm.at[src_v.at[a]], buf0, sem0).wait()
            pltpu.sync_copy(buf0, acc.at[dst_v.at[a]], add=True)

            @pl.when(j < NCH // 2 - 1)
            def _():
                pltpu.async_copy(table_hbm.at[src_v.at[a + 2]], buf0, sem0)

            pltpu.make_async_copy(table_hbm.at[src_v.at[b]], buf1, sem1).wait()
            pltpu.sync_copy(buf1, acc.at[dst_v.at[b]], add=True)
            return carry

        lax.fori_loop(0, NCH // 2, body, 0)

    plsc.subcore_barrier()
    sl = pl.ds(s * RPT, RPT)
    pltpu.sync_copy(acc.at[sl], out_hbm.at[c, sl])


@functools.lru_cache(maxsize=None)
def _get_mp_call(tn, rounds):
    return pl.kernel(
        functools.partial(_mp_body, rounds=rounds),
        out_type=jax.ShapeDtypeStruct((NSC, NPAD, 128), jnp.float32),
        mesh=_get_mesh(),
        compiler_params=pltpu.CompilerParams(use_tc_tiling_on_sc=False),
        scratch_types=[
            pltpu.VMEM((NCH, K), jnp.int32),
            pltpu.VMEM((NCH, K), jnp.int32),
            pltpu.VMEM((K, 128), jnp.float32),
            pltpu.VMEM((K, 128), jnp.float32),
            pltpu.SemaphoreType.DMA,
            pltpu.SemaphoreType.DMA,
            pltpu.VMEM_SHARED((NPAD, 128), jnp.float32),
        ],
        name=f"mp_{tn}",
    )


_BLK = 2000
_NBLK = N // _BLK


def _rsqrt_clip(deg):
    return lax.rsqrt(jnp.maximum(deg, 1.0))


def _tc_scale_body(x_ref, od_ref, o_ref):
    p = _rsqrt_clip(od_ref[0] + od_ref[1])
    o_ref[...] = x_ref[...] * p


def _tc_scale(x, od2):
    return pl.pallas_call(
        _tc_scale_body,
        grid=(_NBLK,),
        in_specs=[
            pl.BlockSpec((_BLK, D_IN), lambda i: (i, 0)),
            pl.BlockSpec((NSC, _BLK, 1), lambda i: (0, i, 0)),
        ],
        out_specs=pl.BlockSpec((_BLK, D_IN), lambda i: (i, 0)),
        out_shape=jax.ShapeDtypeStruct((N, D_IN), jnp.float32),
    )(x, od2)


def _tc_layer1_body(acc_ref, od_ref, id_ref, w0_ref, b0_ref, o_ref):
    a = acc_ref[0] + acc_ref[1]
    m = jnp.dot(a, w0_ref[...], preferred_element_type=jnp.float32)
    q = _rsqrt_clip(id_ref[0] + id_ref[1])
    h = jnp.maximum(q * m + b0_ref[...], 0.0)
    z = h * _rsqrt_clip(od_ref[0] + od_ref[1])
    o_ref[0] = z[:, :128]
    o_ref[1] = z[:, 128:]


def _tc_layer1(acc1, od2, id2, w0, b0):
    return pl.pallas_call(
        _tc_layer1_body,
        grid=(_NBLK,),
        in_specs=[
            pl.BlockSpec((NSC, _BLK, D_IN), lambda i: (0, i, 0)),
            pl.BlockSpec((NSC, _BLK, 1), lambda i: (0, i, 0)),
            pl.BlockSpec((NSC, _BLK, 1), lambda i: (0, i, 0)),
            pl.BlockSpec((D_IN, D_H), lambda i: (0, 0)),
            pl.BlockSpec((1, D_H), lambda i: (0, 0)),
        ],
        out_specs=pl.BlockSpec((NSC, _BLK, 128), lambda i: (0, i, 0)),
        out_shape=jax.ShapeDtypeStruct((NSC, N, 128), jnp.float32),
    )(acc1, od2, id2, w0, b0)


def _tc_head_body(acc_ref, id_ref, w1_ref, b1_ref, ap_ref, wl_ref,
                  bl_ref, o_ref, accum):
    j = pl.program_id(0)

    @pl.when(j == 0)
    def _():
        accum[...] = jnp.zeros_like(accum)

    m = jnp.dot(acc_ref[0], w1_ref[:128, :], preferred_element_type=jnp.float32)
    m += jnp.dot(acc_ref[1], w1_ref[128:, :], preferred_element_type=jnp.float32)
    q = _rsqrt_clip(id_ref[0] + id_ref[1])
    h = jnp.maximum(q * m + b1_ref[...], 0.0)
    accum[...] += jnp.sum(h, axis=0, keepdims=True)

    @pl.when(j == _NBLK - 1)
    def _():
        g = accum[...] * (1.0 / N)
        g = jnp.where(g >= 0.0, g, ap_ref[0, 0] * g)
        v = jnp.dot(g, wl_ref[...], preferred_element_type=jnp.float32)
        o_ref[...] = 1.0 / (1.0 + jnp.exp(-(v + bl_ref[...])))


def _tc_head(acc2, id2, w1, b1, ap, wl, bl):
    return pl.pallas_call(
        _tc_head_body,
        grid=(_NBLK,),
        in_specs=[
            pl.BlockSpec((NSC, _BLK, 128), lambda i: (0, i, 0)),
            pl.BlockSpec((NSC, _BLK, 1), lambda i: (0, i, 0)),
            pl.BlockSpec((D_H, D_H), lambda i: (0, 0)),
            pl.BlockSpec((1, D_H), lambda i: (0, 0)),
            pl.BlockSpec((1, 1), lambda i: (0, 0)),
            pl.BlockSpec((D_H, 1), lambda i: (0, 0)),
            pl.BlockSpec((1, 1), lambda i: (0, 0)),
        ],
        out_specs=pl.BlockSpec((1, 1), lambda i: (0, 0)),
        out_shape=jax.ShapeDtypeStruct((1, 1), jnp.float32),
        scratch_shapes=[pltpu.VMEM((1, D_H), jnp.float32)],
    )(acc2, id2, w1, b1, ap, wl, bl)


@jax.jit
def kernel(x, edge_index, W0, b0, W1, b1, a_prelu, Wl, bl):
    src = edge_index[0]
    dst = edge_index[1]

    srcf = src.reshape(NSC, NT, EPT)
    dstf = dst.reshape(NSC, NT, EPT)
    od, idg = _get_deg_call()(srcf, dstf)
    od2 = od.reshape(NSC, NPAD, 1)
    id2 = idg.reshape(NSC, NPAD, 1)

    srcr1 = src.reshape(NSC, NT, 1, NCH, K)
    dstr1 = dst.reshape(NSC, NT, 1, NCH, K)
    sb = src.reshape(NT, 2, NCH, K)
    db = dst.reshape(NT, 2, NCH, K)
    srcr2 = jnp.stack([sb, sb + N])
    dstr2 = jnp.stack([db, db])
    zeros128 = jnp.zeros((RPT, 128), jnp.float32)

    xs = _tc_scale(x, od2)
    acc1 = _get_mp_call("l1", 1)(xs, srcr1, dstr1, zeros128)
    z = _tc_layer1(acc1, od2, id2, W0, b0.reshape(1, D_H))
    acc2 = _get_mp_call("l2", 2)(z.reshape(NSC * N, 128), srcr2, dstr2,
                                 zeros128)
    out = _tc_head(acc2, id2, W1, b1.reshape(1, D_H),
                   a_prelu.reshape(1, 1), Wl, bl.reshape(1, 1))
    return out


# speedup vs baseline: 22.8019x; 1.0424x over previous
import functools

import jax
import jax.numpy as jnp
from jax import lax
from jax.experimental import pallas as pl
from jax.experimental.pallas import tpu as pltpu
from jax.experimental.pallas import tpu_sc as plsc

N = 10000
E = 320000
D_IN = 128
D_H = 256

NSC = 2
NT = 16
K = 100
NPAD = 10240
RPT = NPAD // NT
EPT = E // (NSC * NT)


@functools.lru_cache(maxsize=None)
def _get_mesh():
    return plsc.VectorSubcoreMesh(core_axis_name="c", subcore_axis_name="s")


def _deg_body(src_hbm, dst_hbm, ood_hbm, oid_hbm, sidx, didx, od_l, id_l, rbuf,
              obuf, sh):
    c = lax.axis_index("c")
    s = lax.axis_index("s")
    pltpu.sync_copy(src_hbm.at[c, s], sidx)
    pltpu.sync_copy(dst_hbm.at[c, s], didx)

    zeros = jnp.zeros((16,), jnp.float32)
    ones = jnp.ones((16,), jnp.float32)

    def zbody(i, carry):
        od_l[pl.ds(i * 16, 16)] = zeros
        id_l[pl.ds(i * 16, 16)] = zeros
        return carry

    lax.fori_loop(0, NPAD // 16, zbody, 0)

    def hbody(i, carry):
        plsc.addupdate_scatter(od_l, [sidx[pl.ds(i * 16, 16)]], ones)
        plsc.addupdate_scatter(id_l, [didx[pl.ds(i * 16, 16)]], ones)
        return carry

    lax.fori_loop(0, EPT // 16, hbody, 0)

    pltpu.sync_copy(od_l, sh.at[s, 0])
    pltpu.sync_copy(id_l, sh.at[s, 1])
    plsc.subcore_barrier()

    for q, out_hbm in ((0, ood_hbm), (1, oid_hbm)):
        for w in range(NT):
            pltpu.sync_copy(sh.at[w, q, pl.ds(s * RPT, RPT)], rbuf.at[w])

        def rbody(i, carry):
            acc = rbuf[0, pl.ds(i * 16, 16)]
            for w in range(1, NT):
                acc += rbuf[w, pl.ds(i * 16, 16)]
            obuf[pl.ds(i * 16, 16)] = acc
            return carry

        lax.fori_loop(0, RPT // 16, rbody, 0)
        pltpu.sync_copy(obuf, out_hbm.at[c, pl.ds(s * RPT, RPT)])


@functools.lru_cache(maxsize=None)
def _get_deg_call():
    return pl.kernel(
        _deg_body,
        out_type=[jax.ShapeDtypeStruct((NSC, NPAD), jnp.float32),
                  jax.ShapeDtypeStruct((NSC, NPAD), jnp.float32)],
        mesh=_get_mesh(),
        compiler_params=pltpu.CompilerParams(needs_layout_passes=False),
        scratch_types=[
            pltpu.VMEM((EPT,), jnp.int32),
            pltpu.VMEM((EPT,), jnp.int32),
            pltpu.VMEM((NPAD,), jnp.float32),
            pltpu.VMEM((NPAD,), jnp.float32),
            pltpu.VMEM((NT, RPT), jnp.float32),
            pltpu.VMEM((RPT,), jnp.float32),
            pltpu.VMEM_SHARED((NT, 2, NPAD), jnp.float32),
        ],
    )


NCH = 100


def _mp_body(table_hbm, srcr_hbm, dstr_hbm, zeros_hbm, out_hbm,
             src_v, dst_v, buf0, buf1, sem0, sem1, acc, *, rounds):
    c = lax.axis_index("c")
    s = lax.axis_index("s")
    pltpu.sync_copy(zeros_hbm, acc.at[pl.ds(s * RPT, RPT)])
    plsc.subcore_barrier()

    for r in range(rounds):
        pltpu.sync_copy(srcr_hbm.at[c, s, r], src_v)
        pltpu.sync_copy(dstr_hbm.at[c, s, r], dst_v)
        pltpu.async_copy(table_hbm.at[src_v.at[0]], buf0, sem0)

        def body(j, carry):
            a = 2 * j
            b = 2 * j + 1
            pltpu.async_copy(table_hbm.at[src_v.at[b]], buf1, sem1)
            pltpu.make_async_copy(table_hbm.at[src_v.at[a]], buf0, sem0).wait()
            pltpu.sync_copy(buf0, acc.at[dst_v.at[a]], add=True)

            @pl.when(j < NCH // 2 - 1)
            def _():
                pltpu.async_copy(table_hbm.at[src_v.at[a + 2]], buf0, sem0)

            pltpu.make_async_copy(table_hbm.at[src_v.at[b]], buf1, sem1).wait()
            pltpu.sync_copy(buf1, acc.at[dst_v.at[b]], add=True)
            return carry

        lax.fori_loop(0, NCH // 2, body, 0)

    plsc.subcore_barrier()
    sl = pl.ds(s * RPT, RPT)
    pltpu.sync_copy(acc.at[sl], out_hbm.at[c, sl])


@functools.lru_cache(maxsize=None)
def _get_mp_call(tn, rounds):
    return pl.kernel(
        functools.partial(_mp_body, rounds=rounds),
        out_type=jax.ShapeDtypeStruct((NSC, NPAD, 128), jnp.float32),
        mesh=_get_mesh(),
        compiler_params=pltpu.CompilerParams(use_tc_tiling_on_sc=False),
        scratch_types=[
            pltpu.VMEM((NCH, K), jnp.int32),
            pltpu.VMEM((NCH, K), jnp.int32),
            pltpu.VMEM((K, 128), jnp.float32),
            pltpu.VMEM((K, 128), jnp.float32),
            pltpu.SemaphoreType.DMA,
            pltpu.SemaphoreType.DMA,
            pltpu.VMEM_SHARED((NPAD, 128), jnp.float32),
        ],
        name=f"mp_{tn}",
    )


_BLK = 1280
_NBLK = NPAD // _BLK


def _rsqrt_clip(deg):
    return lax.rsqrt(jnp.maximum(deg, 1.0))


def _tc_scale_body(x_ref, od_ref, o_ref):
    od = od_ref[0] + od_ref[1]
    p = _rsqrt_clip(od)[:, None]
    o_ref[...] = x_ref[...] * p


def _tc_scale(x, od):
    return pl.pallas_call(
        _tc_scale_body,
        grid=(_NBLK,),
        in_specs=[
            pl.BlockSpec((_BLK, D_IN), lambda i: (i, 0)),
            pl.BlockSpec((NSC, _BLK), lambda i: (0, i)),
        ],
        out_specs=pl.BlockSpec((_BLK, D_IN), lambda i: (i, 0)),
        out_shape=jax.ShapeDtypeStruct((NPAD, D_IN), jnp.float32),
    )(x, od)


def _tc_layer1_body(acc_ref, od_ref, id_ref, w0_ref, b0_ref, o_ref):
    a = acc_ref[0] + acc_ref[1]
    m = jnp.dot(a, w0_ref[...], preferred_element_type=jnp.float32)
    q = _rsqrt_clip(id_ref[0] + id_ref[1])[:, None]
    h = jnp.maximum(q * m + b0_ref[...], 0.0)
    z = h * _rsqrt_clip(od_ref[0] + od_ref[1])[:, None]
    o_ref[0] = z[:, :128]
    o_ref[1] = z[:, 128:]


def _tc_layer1(acc1, od, idg, w0, b0):
    return pl.pallas_call(
        _tc_layer1_body,
        grid=(_NBLK,),
        in_specs=[
            pl.BlockSpec((NSC, _BLK, D_IN), lambda i: (0, i, 0)),
            pl.BlockSpec((NSC, _BLK), lambda i: (0, i)),
            pl.BlockSpec((NSC, _BLK), lambda i: (0, i)),
            pl.BlockSpec((D_IN, D_H), lambda i: (0, 0)),
            pl.BlockSpec((1, D_H), lambda i: (0, 0)),
        ],
        out_specs=pl.BlockSpec((NSC, _BLK, 128), lambda i: (0, i, 0)),
        out_shape=jax.ShapeDtypeStruct((NSC, NPAD, 128), jnp.float32),
    )(acc1, od, idg, w0, b0)


def _tc_head_body(acc_ref, id_ref, w1_ref, b1_ref, ap_ref, wl_ref,
                  bl_ref, o_ref, accum):
    j = pl.program_id(0)

    @pl.when(j == 0)
    def _():
        accum[...] = jnp.zeros_like(accum)

    m = jnp.dot(acc_ref[0], w1_ref[:128, :], preferred_element_type=jnp.float32)
    m += jnp.dot(acc_ref[1], w1_ref[128:, :], preferred_element_type=jnp.float32)
    q = _rsqrt_clip(id_ref[0] + id_ref[1])[:, None]
    h = jnp.maximum(q * m + b1_ref[...], 0.0)
    rows = jax.lax.broadcasted_iota(jnp.int32, (_BLK, 1), 0) + j * _BLK
    h = jnp.where(rows < N, h, 0.0)
    accum[...] += jnp.sum(h, axis=0, keepdims=True)

    @pl.when(j == _NBLK - 1)
    def _():
        g = accum[...] * (1.0 / N)
        g = jnp.where(g >= 0.0, g, ap_ref[0, 0] * g)
        v = jnp.dot(g, wl_ref[...], preferred_element_type=jnp.float32)
        o_ref[...] = 1.0 / (1.0 + jnp.exp(-(v + bl_ref[...])))


def _tc_head(acc2, idg, w1, b1, ap, wl, bl):
    return pl.pallas_call(
        _tc_head_body,
        grid=(_NBLK,),
        in_specs=[
            pl.BlockSpec((NSC, _BLK, 128), lambda i: (0, i, 0)),
            pl.BlockSpec((NSC, _BLK), lambda i: (0, i)),
            pl.BlockSpec((D_H, D_H), lambda i: (0, 0)),
            pl.BlockSpec((1, D_H), lambda i: (0, 0)),
            pl.BlockSpec((1, 1), lambda i: (0, 0)),
            pl.BlockSpec((D_H, 1), lambda i: (0, 0)),
            pl.BlockSpec((1, 1), lambda i: (0, 0)),
        ],
        out_specs=pl.BlockSpec((1, 1), lambda i: (0, 0)),
        out_shape=jax.ShapeDtypeStruct((1, 1), jnp.float32),
        scratch_shapes=[pltpu.VMEM((1, D_H), jnp.float32)],
    )(acc2, idg, w1, b1, ap, wl, bl)


@jax.jit
def kernel(x, edge_index, W0, b0, W1, b1, a_prelu, Wl, bl):
    src = edge_index[0]
    dst = edge_index[1]

    srcf = src.reshape(NSC, NT, EPT)
    dstf = dst.reshape(NSC, NT, EPT)
    od, idg = _get_deg_call()(srcf, dstf)

    srcr1 = src.reshape(NSC, NT, 1, NCH, K)
    dstr1 = dst.reshape(NSC, NT, 1, NCH, K)
    sb = src.reshape(NT, 2, NCH, K)
    db = dst.reshape(NT, 2, NCH, K)
    srcr2 = jnp.stack([sb, sb + NPAD])
    dstr2 = jnp.stack([db, db])
    zeros128 = jnp.zeros((RPT, 128), jnp.float32)

    xs = _tc_scale(x, od)
    acc1 = _get_mp_call("l1", 1)(xs, srcr1, dstr1, zeros128)
    z = _tc_layer1(acc1, od, idg, W0, b0.reshape(1, D_H))
    acc2 = _get_mp_call("l2", 2)(z.reshape(NSC * NPAD, 128), srcr2, dstr2,
                                 zeros128)
    out = _tc_head(acc2, idg, W1, b1.reshape(1, D_H),
                   a_prelu.reshape(1, 1), Wl, bl.reshape(1, 1))
    return out


# speedup vs baseline: 23.2173x; 1.0182x over previous
import functools

import jax
import jax.numpy as jnp
from jax import lax
from jax.experimental import pallas as pl
from jax.experimental.pallas import tpu as pltpu
from jax.experimental.pallas import tpu_sc as plsc

N = 10000
E = 320000
D_IN = 128
D_H = 256

NSC = 2
NT = 16
K = 100
NPAD = 10240
RPT = NPAD // NT
EPT = E // (NSC * NT)


@functools.lru_cache(maxsize=None)
def _get_mesh():
    return plsc.VectorSubcoreMesh(core_axis_name="c", subcore_axis_name="s")


def _deg_body(ei_hbm, ood_hbm, oid_hbm, sidx, didx, od_l, id_l, rbuf,
              obuf, sh):
    c = lax.axis_index("c")
    s = lax.axis_index("s")
    w = c * NT + s
    pltpu.sync_copy(ei_hbm.at[pl.ds(w * EPT, EPT)], sidx)
    pltpu.sync_copy(ei_hbm.at[pl.ds(E + w * EPT, EPT)], didx)

    zeros = jnp.zeros((16,), jnp.float32)
    ones = jnp.ones((16,), jnp.float32)

    def zbody(i, carry):
        od_l[pl.ds(i * 16, 16)] = zeros
        id_l[pl.ds(i * 16, 16)] = zeros
        return carry

    lax.fori_loop(0, NPAD // 16, zbody, 0)

    def hbody(i, carry):
        plsc.addupdate_scatter(od_l, [sidx[pl.ds(i * 16, 16)]], ones)
        plsc.addupdate_scatter(id_l, [didx[pl.ds(i * 16, 16)]], ones)
        return carry

    lax.fori_loop(0, EPT // 16, hbody, 0)

    pltpu.sync_copy(od_l, sh.at[s, 0])
    pltpu.sync_copy(id_l, sh.at[s, 1])
    plsc.subcore_barrier()

    for q, out_hbm in ((0, ood_hbm), (1, oid_hbm)):
        for w in range(NT):
            pltpu.sync_copy(sh.at[w, q, pl.ds(s * RPT, RPT)], rbuf.at[w])

        def rbody(i, carry):
            acc = rbuf[0, pl.ds(i * 16, 16)]
            for w in range(1, NT):
                acc += rbuf[w, pl.ds(i * 16, 16)]
            obuf[pl.ds(i * 16, 16)] = acc
            return carry

        lax.fori_loop(0, RPT // 16, rbody, 0)
        pltpu.sync_copy(obuf, out_hbm.at[c, pl.ds(s * RPT, RPT)])


@functools.lru_cache(maxsize=None)
def _get_deg_call():
    return pl.kernel(
        _deg_body,
        out_type=[jax.ShapeDtypeStruct((NSC, NPAD), jnp.float32),
                  jax.ShapeDtypeStruct((NSC, NPAD), jnp.float32)],
        mesh=_get_mesh(),
        compiler_params=pltpu.CompilerParams(needs_layout_passes=False),
        scratch_types=[
            pltpu.VMEM((EPT,), jnp.int32),
            pltpu.VMEM((EPT,), jnp.int32),
            pltpu.VMEM((NPAD,), jnp.float32),
            pltpu.VMEM((NPAD,), jnp.float32),
            pltpu.VMEM((NT, RPT), jnp.float32),
            pltpu.VMEM((RPT,), jnp.float32),
            pltpu.VMEM_SHARED((NT, 2, NPAD), jnp.float32),
        ],
    )


NCH = 100


def _mp_body(table_hbm, srcr_hbm, dstr_hbm, zeros_hbm, out_hbm,
             src_v, dst_v, buf0, buf1, sem0, sem1, acc, *, rounds):
    c = lax.axis_index("c")
    s = lax.axis_index("s")
    pltpu.sync_copy(zeros_hbm, acc.at[pl.ds(s * RPT, RPT)])
    plsc.subcore_barrier()

    for r in range(rounds):
        pltpu.sync_copy(srcr_hbm.at[c, s, r], src_v)
        pltpu.sync_copy(dstr_hbm.at[c, s, r], dst_v)
        pltpu.async_copy(table_hbm.at[src_v.at[0]], buf0, sem0)

        def body(j, carry):
            a = 2 * j
            b = 2 * j + 1
            pltpu.async_copy(table_hbm.at[src_v.at[b]], buf1, sem1)
            pltpu.make_async_copy(table_hbm.at[src_v.at[a]], buf0, sem0).wait()
            pltpu.sync_copy(buf0, acc.at[dst_v.at[a]], add=True)

            @pl.when(j < NCH // 2 - 1)
            def _():
                pltpu.async_copy(table_hbm.at[src_v.at[a + 2]], buf0, sem0)

            pltpu.make_async_copy(table_hbm.at[src_v.at[b]], buf1, sem1).wait()
            pltpu.sync_copy(buf1, acc.at[dst_v.at[b]], add=True)
            return carry

        lax.fori_loop(0, NCH // 2, body, 0)

    plsc.subcore_barrier()
    sl = pl.ds(s * RPT, RPT)
    pltpu.sync_copy(acc.at[sl], out_hbm.at[c, sl])


@functools.lru_cache(maxsize=None)
def _get_mp_call(tn, rounds):
    return pl.kernel(
        functools.partial(_mp_body, rounds=rounds),
        out_type=jax.ShapeDtypeStruct((NSC, NPAD, 128), jnp.float32),
        mesh=_get_mesh(),
        compiler_params=pltpu.CompilerParams(use_tc_tiling_on_sc=False),
        scratch_types=[
            pltpu.VMEM((NCH, K), jnp.int32),
            pltpu.VMEM((NCH, K), jnp.int32),
            pltpu.VMEM((K, 128), jnp.float32),
            pltpu.VMEM((K, 128), jnp.float32),
            pltpu.SemaphoreType.DMA,
            pltpu.SemaphoreType.DMA,
            pltpu.VMEM_SHARED((NPAD, 128), jnp.float32),
        ],
        name=f"mp_{tn}",
    )


_BLK = 1280
_NBLK = NPAD // _BLK


def _rsqrt_clip(deg):
    return lax.rsqrt(jnp.maximum(deg, 1.0))


def _tc_scale_body(x_ref, od_ref, o_ref):
    od = od_ref[0] + od_ref[1]
    p = _rsqrt_clip(od)[:, None]
    o_ref[...] = x_ref[...] * p


def _tc_scale(x, od):
    return pl.pallas_call(
        _tc_scale_body,
        grid=(_NBLK,),
        in_specs=[
            pl.BlockSpec((_BLK, D_IN), lambda i: (i, 0)),
            pl.BlockSpec((NSC, _BLK), lambda i: (0, i)),
        ],
        out_specs=pl.BlockSpec((_BLK, D_IN), lambda i: (i, 0)),
        out_shape=jax.ShapeDtypeStruct((NPAD, D_IN), jnp.float32),
    )(x, od)


def _tc_layer1_body(acc_ref, od_ref, id_ref, w0_ref, b0_ref, o_ref):
    a = acc_ref[0] + acc_ref[1]
    m = jnp.dot(a, w0_ref[...], preferred_element_type=jnp.float32)
    q = _rsqrt_clip(id_ref[0] + id_ref[1])[:, None]
    h = jnp.maximum(q * m + b0_ref[...], 0.0)
    z = h * _rsqrt_clip(od_ref[0] + od_ref[1])[:, None]
    o_ref[0] = z[:, :128]
    o_ref[1] = z[:, 128:]


def _tc_layer1(acc1, od, idg, w0, b0):
    return pl.pallas_call(
        _tc_layer1_body,
        grid=(_NBLK,),
        in_specs=[
            pl.BlockSpec((NSC, _BLK, D_IN), lambda i: (0, i, 0)),
            pl.BlockSpec((NSC, _BLK), lambda i: (0, i)),
            pl.BlockSpec((NSC, _BLK), lambda i: (0, i)),
            pl.BlockSpec((D_IN, D_H), lambda i: (0, 0)),
            pl.BlockSpec((1, D_H), lambda i: (0, 0)),
        ],
        out_specs=pl.BlockSpec((NSC, _BLK, 128), lambda i: (0, i, 0)),
        out_shape=jax.ShapeDtypeStruct((NSC, NPAD, 128), jnp.float32),
    )(acc1, od, idg, w0, b0)


def _tc_head_body(acc_ref, id_ref, w1_ref, b1_ref, ap_ref, wl_ref,
                  bl_ref, o_ref, accum):
    j = pl.program_id(0)

    @pl.when(j == 0)
    def _():
        accum[...] = jnp.zeros_like(accum)

    m = jnp.dot(acc_ref[0], w1_ref[:128, :], preferred_element_type=jnp.float32)
    m += jnp.dot(acc_ref[1], w1_ref[128:, :], preferred_element_type=jnp.float32)
    q = _rsqrt_clip(id_ref[0] + id_ref[1])[:, None]
    h = jnp.maximum(q * m + b1_ref[...], 0.0)
    rows = jax.lax.broadcasted_iota(jnp.int32, (_BLK, 1), 0) + j * _BLK
    h = jnp.where(rows < N, h, 0.0)
    accum[...] += jnp.sum(h, axis=0, keepdims=True)

    @pl.when(j == _NBLK - 1)
    def _():
        g = accum[...] * (1.0 / N)
        g = jnp.where(g >= 0.0, g, ap_ref[0, 0] * g)
        v = jnp.dot(g, wl_ref[...], preferred_element_type=jnp.float32)
        o_ref[...] = 1.0 / (1.0 + jnp.exp(-(v + bl_ref[...])))


def _tc_head(acc2, idg, w1, b1, ap, wl, bl):
    return pl.pallas_call(
        _tc_head_body,
        grid=(_NBLK,),
        in_specs=[
            pl.BlockSpec((NSC, _BLK, 128), lambda i: (0, i, 0)),
            pl.BlockSpec((NSC, _BLK), lambda i: (0, i)),
            pl.BlockSpec((D_H, D_H), lambda i: (0, 0)),
            pl.BlockSpec((1, D_H), lambda i: (0, 0)),
            pl.BlockSpec((1, 1), lambda i: (0, 0)),
            pl.BlockSpec((D_H, 1), lambda i: (0, 0)),
            pl.BlockSpec((1, 1), lambda i: (0, 0)),
        ],
        out_specs=pl.BlockSpec((1, 1), lambda i: (0, 0)),
        out_shape=jax.ShapeDtypeStruct((1, 1), jnp.float32),
        scratch_shapes=[pltpu.VMEM((1, D_H), jnp.float32)],
    )(acc2, idg, w1, b1, ap, wl, bl)


@jax.jit
def kernel(x, edge_index, W0, b0, W1, b1, a_prelu, Wl, bl):
    src = edge_index[0]
    dst = edge_index[1]

    ei_flat = edge_index.reshape(NSC * E)
    od, idg = _get_deg_call()(ei_flat)

    srcr1 = src.reshape(NSC, NT, 1, NCH, K)
    dstr1 = dst.reshape(NSC, NT, 1, NCH, K)
    sb = src.reshape(NT, 2, NCH, K)
    db = dst.reshape(NT, 2, NCH, K)
    srcr2 = jnp.stack([sb, sb + NPAD])
    dstr2 = jnp.stack([db, db])
    zeros128 = jnp.zeros((RPT, 128), jnp.float32)

    xs = _tc_scale(x, od)
    acc1 = _get_mp_call("l1", 1)(xs, srcr1, dstr1, zeros128)
    z = _tc_layer1(acc1, od, idg, W0, b0.reshape(1, D_H))
    acc2 = _get_mp_call("l2", 2)(z.reshape(NSC * NPAD, 128), srcr2, dstr2,
                                 zeros128)
    out = _tc_head(acc2, idg, W1, b1.reshape(1, D_H),
                   a_prelu.reshape(1, 1), Wl, bl.reshape(1, 1))
    return out


# speedup vs baseline: 25.0353x; 1.0783x over previous
import functools

import jax
import jax.numpy as jnp
from jax import lax
from jax.experimental import pallas as pl
from jax.experimental.pallas import tpu as pltpu
from jax.experimental.pallas import tpu_sc as plsc

N = 10000
E = 320000
D_IN = 128
D_H = 256

NSC = 2
NT = 16
K = 50
NB = 4
NPAD = 10240
RPT = NPAD // NT
EPT = E // (NSC * NT)


@functools.lru_cache(maxsize=None)
def _get_mesh():
    return plsc.VectorSubcoreMesh(core_axis_name="c", subcore_axis_name="s")


def _deg_body(ei_hbm, ood_hbm, oid_hbm, sidx, didx, od_l, id_l, rbuf,
              obuf, sh):
    c = lax.axis_index("c")
    s = lax.axis_index("s")
    w = c * NT + s
    pltpu.sync_copy(ei_hbm.at[pl.ds(w * EPT, EPT)], sidx)
    pltpu.sync_copy(ei_hbm.at[pl.ds(E + w * EPT, EPT)], didx)

    zeros = jnp.zeros((16,), jnp.float32)
    ones = jnp.ones((16,), jnp.float32)

    def zbody(i, carry):
        od_l[pl.ds(i * 16, 16)] = zeros
        id_l[pl.ds(i * 16, 16)] = zeros
        return carry

    lax.fori_loop(0, NPAD // 16, zbody, 0)

    def hbody(i, carry):
        plsc.addupdate_scatter(od_l, [sidx[pl.ds(i * 16, 16)]], ones)
        plsc.addupdate_scatter(id_l, [didx[pl.ds(i * 16, 16)]], ones)
        return carry

    lax.fori_loop(0, EPT // 16, hbody, 0)

    pltpu.sync_copy(od_l, sh.at[s, 0])
    pltpu.sync_copy(id_l, sh.at[s, 1])
    plsc.subcore_barrier()

    for q, out_hbm in ((0, ood_hbm), (1, oid_hbm)):
        for w in range(NT):
            pltpu.sync_copy(sh.at[w, q, pl.ds(s * RPT, RPT)], rbuf.at[w])

        def rbody(i, carry):
            acc = rbuf[0, pl.ds(i * 16, 16)]
            for w in range(1, NT):
                acc += rbuf[w, pl.ds(i * 16, 16)]
            obuf[pl.ds(i * 16, 16)] = acc
            return carry

        lax.fori_loop(0, RPT // 16, rbody, 0)
        pltpu.sync_copy(obuf, out_hbm.at[c, pl.ds(s * RPT, RPT)])


@functools.lru_cache(maxsize=None)
def _get_deg_call():
    return pl.kernel(
        _deg_body,
        out_type=[jax.ShapeDtypeStruct((NSC, NPAD), jnp.float32),
                  jax.ShapeDtypeStruct((NSC, NPAD), jnp.float32)],
        mesh=_get_mesh(),
        compiler_params=pltpu.CompilerParams(needs_layout_passes=False),
        scratch_types=[
            pltpu.VMEM((EPT,), jnp.int32),
            pltpu.VMEM((EPT,), jnp.int32),
            pltpu.VMEM((NPAD,), jnp.float32),
            pltpu.VMEM((NPAD,), jnp.float32),
            pltpu.VMEM((NT, RPT), jnp.float32),
            pltpu.VMEM((RPT,), jnp.float32),
            pltpu.VMEM_SHARED((NT, 2, NPAD), jnp.float32),
        ],
    )


NCH = 10000 // K


def _mp_body(table_hbm, srcr_hbm, dstr_hbm, zeros_hbm, out_hbm,
             src_v, dst_v, bufs, sems, acc, *, rounds):
    c = lax.axis_index("c")
    s = lax.axis_index("s")
    pltpu.sync_copy(zeros_hbm, acc.at[pl.ds(s * RPT, RPT)])
    plsc.subcore_barrier()

    for r in range(rounds):
        pltpu.sync_copy(srcr_hbm.at[c, s, r], src_v)
        pltpu.sync_copy(dstr_hbm.at[c, s, r], dst_v)
        for t in range(NB):
            pltpu.async_copy(table_hbm.at[src_v.at[t]], bufs[t], sems[t])

        def body(j, carry):
            for t in range(NB):
                ch = NB * j + t
                pltpu.make_async_copy(
                    table_hbm.at[src_v.at[ch]], bufs[t], sems[t]).wait()
                pltpu.sync_copy(bufs[t], acc.at[dst_v.at[ch]], add=True)

                @pl.when(j < NCH // NB - 1)
                def _():
                    pltpu.async_copy(
                        table_hbm.at[src_v.at[ch + NB]], bufs[t], sems[t])

            return carry

        lax.fori_loop(0, NCH // NB, body, 0)

    plsc.subcore_barrier()
    sl = pl.ds(s * RPT, RPT)
    pltpu.sync_copy(acc.at[sl], out_hbm.at[c, sl])


@functools.lru_cache(maxsize=None)
def _get_mp_call(tn, rounds):
    return pl.kernel(
        functools.partial(_mp_body, rounds=rounds),
        out_type=jax.ShapeDtypeStruct((NSC, NPAD, 128), jnp.float32),
        mesh=_get_mesh(),
        compiler_params=pltpu.CompilerParams(use_tc_tiling_on_sc=False),
        scratch_types=[
            pltpu.VMEM((NCH, K), jnp.int32),
            pltpu.VMEM((NCH, K), jnp.int32),
            [pltpu.VMEM((K, 128), jnp.float32) for _ in range(NB)],
            [pltpu.SemaphoreType.DMA for _ in range(NB)],
            pltpu.VMEM_SHARED((NPAD, 128), jnp.float32),
        ],
        name=f"mp_{tn}",
    )


_BLK = 1280
_NBLK = NPAD // _BLK


def _rsqrt_clip(deg):
    return lax.rsqrt(jnp.maximum(deg, 1.0))


def _tc_scale_body(x_ref, od_ref, o_ref):
    od = od_ref[0] + od_ref[1]
    p = _rsqrt_clip(od)[:, None]
    o_ref[...] = x_ref[...] * p


def _tc_scale(x, od):
    return pl.pallas_call(
        _tc_scale_body,
        grid=(_NBLK,),
        in_specs=[
            pl.BlockSpec((_BLK, D_IN), lambda i: (i, 0)),
            pl.BlockSpec((NSC, _BLK), lambda i: (0, i)),
        ],
        out_specs=pl.BlockSpec((_BLK, D_IN), lambda i: (i, 0)),
        out_shape=jax.ShapeDtypeStruct((NPAD, D_IN), jnp.float32),
    )(x, od)


def _tc_layer1_body(acc_ref, od_ref, id_ref, w0_ref, b0_ref, o_ref):
    a = acc_ref[0] + acc_ref[1]
    m = jnp.dot(a, w0_ref[...], preferred_element_type=jnp.float32)
    q = _rsqrt_clip(id_ref[0] + id_ref[1])[:, None]
    h = jnp.maximum(q * m + b0_ref[...], 0.0)
    z = h * _rsqrt_clip(od_ref[0] + od_ref[1])[:, None]
    o_ref[0] = z[:, :128]
    o_ref[1] = z[:, 128:]


def _tc_layer1(acc1, od, idg, w0, b0):
    return pl.pallas_call(
        _tc_layer1_body,
        grid=(_NBLK,),
        in_specs=[
            pl.BlockSpec((NSC, _BLK, D_IN), lambda i: (0, i, 0)),
            pl.BlockSpec((NSC, _BLK), lambda i: (0, i)),
            pl.BlockSpec((NSC, _BLK), lambda i: (0, i)),
            pl.BlockSpec((D_IN, D_H), lambda i: (0, 0)),
            pl.BlockSpec((1, D_H), lambda i: (0, 0)),
        ],
        out_specs=pl.BlockSpec((NSC, _BLK, 128), lambda i: (0, i, 0)),
        out_shape=jax.ShapeDtypeStruct((NSC, NPAD, 128), jnp.float32),
    )(acc1, od, idg, w0, b0)


def _tc_head_body(acc_ref, id_ref, w1_ref, b1_ref, ap_ref, wl_ref,
                  bl_ref, o_ref, accum):
    j = pl.program_id(0)

    @pl.when(j == 0)
    def _():
        accum[...] = jnp.zeros_like(accum)

    m = jnp.dot(acc_ref[0], w1_ref[:128, :], preferred_element_type=jnp.float32)
    m += jnp.dot(acc_ref[1], w1_ref[128:, :], preferred_element_type=jnp.float32)
    q = _rsqrt_clip(id_ref[0] + id_ref[1])[:, None]
    h = jnp.maximum(q * m + b1_ref[...], 0.0)
    rows = jax.lax.broadcasted_iota(jnp.int32, (_BLK, 1), 0) + j * _BLK
    h = jnp.where(rows < N, h, 0.0)
    accum[...] += jnp.sum(h, axis=0, keepdims=True)

    @pl.when(j == _NBLK - 1)
    def _():
        g = accum[...] * (1.0 / N)
        g = jnp.where(g >= 0.0, g, ap_ref[0, 0] * g)
        v = jnp.dot(g, wl_ref[...], preferred_element_type=jnp.float32)
        o_ref[...] = 1.0 / (1.0 + jnp.exp(-(v + bl_ref[...])))


def _tc_head(acc2, idg, w1, b1, ap, wl, bl):
    return pl.pallas_call(
        _tc_head_body,
        grid=(_NBLK,),
        in_specs=[
            pl.BlockSpec((NSC, _BLK, 128), lambda i: (0, i, 0)),
            pl.BlockSpec((NSC, _BLK), lambda i: (0, i)),
            pl.BlockSpec((D_H, D_H), lambda i: (0, 0)),
            pl.BlockSpec((1, D_H), lambda i: (0, 0)),
            pl.BlockSpec((1, 1), lambda i: (0, 0)),
            pl.BlockSpec((D_H, 1), lambda i: (0, 0)),
            pl.BlockSpec((1, 1), lambda i: (0, 0)),
        ],
        out_specs=pl.BlockSpec((1, 1), lambda i: (0, 0)),
        out_shape=jax.ShapeDtypeStruct((1, 1), jnp.float32),
        scratch_shapes=[pltpu.VMEM((1, D_H), jnp.float32)],
    )(acc2, idg, w1, b1, ap, wl, bl)


@jax.jit
def kernel(x, edge_index, W0, b0, W1, b1, a_prelu, Wl, bl):
    src = edge_index[0]
    dst = edge_index[1]

    ei_flat = edge_index.reshape(NSC * E)
    od, idg = _get_deg_call()(ei_flat)

    srcr1 = src.reshape(NSC, NT, 1, NCH, K)
    dstr1 = dst.reshape(NSC, NT, 1, NCH, K)
    sb = src.reshape(NT, 2, NCH, K)
    db = dst.reshape(NT, 2, NCH, K)
    srcr2 = jnp.stack([sb, sb + NPAD])
    dstr2 = jnp.stack([db, db])
    zeros128 = jnp.zeros((RPT, 128), jnp.float32)

    xs = _tc_scale(x, od)
    acc1 = _get_mp_call("l1", 1)(xs, srcr1, dstr1, zeros128)
    z = _tc_layer1(acc1, od, idg, W0, b0.reshape(1, D_H))
    acc2 = _get_mp_call("l2", 2)(z.reshape(NSC * NPAD, 128), srcr2, dstr2,
                                 zeros128)
    out = _tc_head(acc2, idg, W1, b1.reshape(1, D_H),
                   a_prelu.reshape(1, 1), Wl, bl.reshape(1, 1))
    return out


# speedup vs baseline: 25.9401x; 1.0361x over previous
import functools

import jax
import jax.numpy as jnp
from jax import lax
from jax.experimental import pallas as pl
from jax.experimental.pallas import tpu as pltpu
from jax.experimental.pallas import tpu_sc as plsc

N = 10000
E = 320000
D_IN = 128
D_H = 256

NSC = 2
NT = 16
K = 40
NB = 5
NPAD = 10240
RPT = NPAD // NT
EPT = E // (NSC * NT)


@functools.lru_cache(maxsize=None)
def _get_mesh():
    return plsc.VectorSubcoreMesh(core_axis_name="c", subcore_axis_name="s")


def _deg_body(ei_hbm, ood_hbm, oid_hbm, sidx, didx, od_l, id_l, rbuf,
              obuf, sh):
    c = lax.axis_index("c")
    s = lax.axis_index("s")
    w = c * NT + s
    pltpu.sync_copy(ei_hbm.at[pl.ds(w * EPT, EPT)], sidx)
    pltpu.sync_copy(ei_hbm.at[pl.ds(E + w * EPT, EPT)], didx)

    zeros = jnp.zeros((16,), jnp.float32)
    ones = jnp.ones((16,), jnp.float32)

    def zbody(i, carry):
        od_l[pl.ds(i * 16, 16)] = zeros
        id_l[pl.ds(i * 16, 16)] = zeros
        return carry

    lax.fori_loop(0, NPAD // 16, zbody, 0)

    def hbody(i, carry):
        plsc.addupdate_scatter(od_l, [sidx[pl.ds(i * 16, 16)]], ones)
        plsc.addupdate_scatter(id_l, [didx[pl.ds(i * 16, 16)]], ones)
        return carry

    lax.fori_loop(0, EPT // 16, hbody, 0)

    pltpu.sync_copy(od_l, sh.at[s, 0])
    pltpu.sync_copy(id_l, sh.at[s, 1])
    plsc.subcore_barrier()

    for q, out_hbm in ((0, ood_hbm), (1, oid_hbm)):
        for w in range(NT):
            pltpu.sync_copy(sh.at[w, q, pl.ds(s * RPT, RPT)], rbuf.at[w])

        def rbody(i, carry):
            acc = rbuf[0, pl.ds(i * 16, 16)]
            for w in range(1, NT):
                acc += rbuf[w, pl.ds(i * 16, 16)]
            obuf[pl.ds(i * 16, 16)] = acc
            return carry

        lax.fori_loop(0, RPT // 16, rbody, 0)
        pltpu.sync_copy(obuf, out_hbm.at[c, pl.ds(s * RPT, RPT)])


@functools.lru_cache(maxsize=None)
def _get_deg_call():
    return pl.kernel(
        _deg_body,
        out_type=[jax.ShapeDtypeStruct((NSC, NPAD), jnp.float32),
                  jax.ShapeDtypeStruct((NSC, NPAD), jnp.float32)],
        mesh=_get_mesh(),
        compiler_params=pltpu.CompilerParams(needs_layout_passes=False),
        scratch_types=[
            pltpu.VMEM((EPT,), jnp.int32),
            pltpu.VMEM((EPT,), jnp.int32),
            pltpu.VMEM((NPAD,), jnp.float32),
            pltpu.VMEM((NPAD,), jnp.float32),
            pltpu.VMEM((NT, RPT), jnp.float32),
            pltpu.VMEM((RPT,), jnp.float32),
            pltpu.VMEM_SHARED((NT, 2, NPAD), jnp.float32),
        ],
    )


NCH = 10000 // K


def _mp_body(table_hbm, srcr_hbm, dstr_hbm, zeros_hbm, out_hbm,
             src_v, dst_v, bufs, sems, acc, *, rounds):
    c = lax.axis_index("c")
    s = lax.axis_index("s")
    pltpu.sync_copy(zeros_hbm, acc.at[pl.ds(s * RPT, RPT)])
    plsc.subcore_barrier()

    for r in range(rounds):
        pltpu.sync_copy(srcr_hbm.at[c, s, r], src_v)
        pltpu.sync_copy(dstr_hbm.at[c, s, r], dst_v)
        for t in range(NB):
            pltpu.async_copy(table_hbm.at[src_v.at[t]], bufs[t], sems[t])

        def body(j, carry):
            for t in range(NB):
                ch = NB * j + t
                pltpu.make_async_copy(
                    table_hbm.at[src_v.at[ch]], bufs[t], sems[t]).wait()
                pltpu.sync_copy(bufs[t], acc.at[dst_v.at[ch]], add=True)

                @pl.when(j < NCH // NB - 1)
                def _():
                    pltpu.async_copy(
                        table_hbm.at[src_v.at[ch + NB]], bufs[t], sems[t])

            return carry

        lax.fori_loop(0, NCH // NB, body, 0)

    plsc.subcore_barrier()
    sl = pl.ds(s * RPT, RPT)
    pltpu.sync_copy(acc.at[sl], out_hbm.at[c, sl])


@functools.lru_cache(maxsize=None)
def _get_mp_call(tn, rounds):
    return pl.kernel(
        functools.partial(_mp_body, rounds=rounds),
        out_type=jax.ShapeDtypeStruct((NSC, NPAD, 128), jnp.float32),
        mesh=_get_mesh(),
        compiler_params=pltpu.CompilerParams(use_tc_tiling_on_sc=False),
        scratch_types=[
            pltpu.VMEM((NCH, K), jnp.int32),
            pltpu.VMEM((NCH, K), jnp.int32),
            [pltpu.VMEM((K, 128), jnp.float32) for _ in range(NB)],
            [pltpu.SemaphoreType.DMA for _ in range(NB)],
            pltpu.VMEM_SHARED((NPAD, 128), jnp.float32),
        ],
        name=f"mp_{tn}",
    )


_BLK = 1280
_NBLK = NPAD // _BLK


def _rsqrt_clip(deg):
    return lax.rsqrt(jnp.maximum(deg, 1.0))


def _tc_scale_body(x_ref, od_ref, o_ref):
    od = od_ref[0] + od_ref[1]
    p = _rsqrt_clip(od)[:, None]
    o_ref[...] = x_ref[...] * p


def _tc_scale(x, od):
    return pl.pallas_call(
        _tc_scale_body,
        grid=(_NBLK,),
        in_specs=[
            pl.BlockSpec((_BLK, D_IN), lambda i: (i, 0)),
            pl.BlockSpec((NSC, _BLK), lambda i: (0, i)),
        ],
        out_specs=pl.BlockSpec((_BLK, D_IN), lambda i: (i, 0)),
        out_shape=jax.ShapeDtypeStruct((NPAD, D_IN), jnp.float32),
    )(x, od)


def _tc_layer1_body(acc_ref, od_ref, id_ref, w0_ref, b0_ref, o_ref):
    a = acc_ref[0] + acc_ref[1]
    m = jnp.dot(a, w0_ref[...], preferred_element_type=jnp.float32)
    q = _rsqrt_clip(id_ref[0] + id_ref[1])[:, None]
    h = jnp.maximum(q * m + b0_ref[...], 0.0)
    z = h * _rsqrt_clip(od_ref[0] + od_ref[1])[:, None]
    o_ref[0] = z[:, :128]
    o_ref[1] = z[:, 128:]


def _tc_layer1(acc1, od, idg, w0, b0):
    return pl.pallas_call(
        _tc_layer1_body,
        grid=(_NBLK,),
        in_specs=[
            pl.BlockSpec((NSC, _BLK, D_IN), lambda i: (0, i, 0)),
            pl.BlockSpec((NSC, _BLK), lambda i: (0, i)),
            pl.BlockSpec((NSC, _BLK), lambda i: (0, i)),
            pl.BlockSpec((D_IN, D_H), lambda i: (0, 0)),
            pl.BlockSpec((1, D_H), lambda i: (0, 0)),
        ],
        out_specs=pl.BlockSpec((NSC, _BLK, 128), lambda i: (0, i, 0)),
        out_shape=jax.ShapeDtypeStruct((NSC, NPAD, 128), jnp.float32),
    )(acc1, od, idg, w0, b0)


def _tc_head_body(acc_ref, id_ref, w1_ref, b1_ref, ap_ref, wl_ref,
                  bl_ref, o_ref, accum):
    j = pl.program_id(0)

    @pl.when(j == 0)
    def _():
        accum[...] = jnp.zeros_like(accum)

    m = jnp.dot(acc_ref[0], w1_ref[:128, :], preferred_element_type=jnp.float32)
    m += jnp.dot(acc_ref[1], w1_ref[128:, :], preferred_element_type=jnp.float32)
    q = _rsqrt_clip(id_ref[0] + id_ref[1])[:, None]
    h = jnp.maximum(q * m + b1_ref[...], 0.0)
    rows = jax.lax.broadcasted_iota(jnp.int32, (_BLK, 1), 0) + j * _BLK
    h = jnp.where(rows < N, h, 0.0)
    accum[...] += jnp.sum(h, axis=0, keepdims=True)

    @pl.when(j == _NBLK - 1)
    def _():
        g = accum[...] * (1.0 / N)
        g = jnp.where(g >= 0.0, g, ap_ref[0, 0] * g)
        v = jnp.dot(g, wl_ref[...], preferred_element_type=jnp.float32)
        o_ref[...] = 1.0 / (1.0 + jnp.exp(-(v + bl_ref[...])))


def _tc_head(acc2, idg, w1, b1, ap, wl, bl):
    return pl.pallas_call(
        _tc_head_body,
        grid=(_NBLK,),
        in_specs=[
            pl.BlockSpec((NSC, _BLK, 128), lambda i: (0, i, 0)),
            pl.BlockSpec((NSC, _BLK), lambda i: (0, i)),
            pl.BlockSpec((D_H, D_H), lambda i: (0, 0)),
            pl.BlockSpec((1, D_H), lambda i: (0, 0)),
            pl.BlockSpec((1, 1), lambda i: (0, 0)),
            pl.BlockSpec((D_H, 1), lambda i: (0, 0)),
            pl.BlockSpec((1, 1), lambda i: (0, 0)),
        ],
        out_specs=pl.BlockSpec((1, 1), lambda i: (0, 0)),
        out_shape=jax.ShapeDtypeStruct((1, 1), jnp.float32),
        scratch_shapes=[pltpu.VMEM((1, D_H), jnp.float32)],
    )(acc2, idg, w1, b1, ap, wl, bl)


@jax.jit
def kernel(x, edge_index, W0, b0, W1, b1, a_prelu, Wl, bl):
    src = edge_index[0]
    dst = edge_index[1]

    ei_flat = edge_index.reshape(NSC * E)
    od, idg = _get_deg_call()(ei_flat)

    srcr1 = src.reshape(NSC, NT, 1, NCH, K)
    dstr1 = dst.reshape(NSC, NT, 1, NCH, K)
    sb = src.reshape(NT, 2, NCH, K)
    db = dst.reshape(NT, 2, NCH, K)
    srcr2 = jnp.stack([sb, sb + NPAD])
    dstr2 = jnp.stack([db, db])
    zeros128 = jnp.zeros((RPT, 128), jnp.float32)

    xs = _tc_scale(x, od)
    acc1 = _get_mp_call("l1", 1)(xs, srcr1, dstr1, zeros128)
    z = _tc_layer1(acc1, od, idg, W0, b0.reshape(1, D_H))
    acc2 = _get_mp_call("l2", 2)(z.reshape(NSC * NPAD, 128), srcr2, dstr2,
                                 zeros128)
    out = _tc_head(acc2, idg, W1, b1.reshape(1, D_H),
                   a_prelu.reshape(1, 1), Wl, bl.reshape(1, 1))
    return out
